# baseline (speedup 1.0000x reference)
import functools

import jax
import jax.numpy as jnp
from jax import lax
from jax.experimental import pallas as pl
from jax.experimental.pallas import tpu as pltpu
from jax.experimental.pallas import tpu_sc as plsc

NC = 2
NS = 16
NW = NC * NS
CHUNK = 128


def _sc_mesh():
    return plsc.VectorSubcoreMesh(
        core_axis_name="c", subcore_axis_name="s", num_cores=NC, num_subcores=NS
    )


def _sc_degree(dst3d, zeros16, npad):
    cpw = dst3d.shape[1]
    half = cpw // 2
    rps = npad // NS

    @functools.partial(
        pl.kernel,
        out_type=jax.ShapeDtypeStruct((NC, npad, 16), jnp.float32),
        mesh=_sc_mesh(),
        scratch_types=[
            pltpu.VMEM((half, CHUNK), jnp.int32),
            pltpu.VMEM((CHUNK, 16), jnp.float32),
            pltpu.VMEM_SHARED((npad, 16), jnp.float32),
        ],
    )
    def deg_kernel(dst_hbm, zeros_hbm, out_hbm, idx_v, ones_v, acc_sh):
        c = lax.axis_index("c")
        s = lax.axis_index("s")
        wid = c * NS + s

        def fill(i, carry):
            ones_v[i, :] = jnp.full((16,), 1.0, jnp.float32)
            return carry

        lax.fori_loop(0, CHUNK, fill, 0)
        pltpu.sync_copy(
            zeros_hbm.at[pl.ds(s * rps, rps)], acc_sh.at[pl.ds(s * rps, rps)]
        )
        plsc.subcore_barrier()

        def body(j, carry):
            pltpu.sync_copy(ones_v, acc_sh.at[idx_v.at[j]], add=True)
            return carry

        for phase in range(2):
            pltpu.sync_copy(dst_hbm.at[wid].at[pl.ds(phase * half, half)], idx_v)
            lax.fori_loop(0, half, body, 0)
        plsc.subcore_barrier()
        pltpu.sync_copy(
            acc_sh.at[pl.ds(s * rps, rps)], out_hbm.at[c, pl.ds(s * rps, rps)]
        )

    return deg_kernel(dst3d, zeros16)


def _sc_aggregate(y, src3d, dst3d, zeros_nd, npad):
    d = y.shape[1]
    cpw = src3d.shape[1]
    half = cpw // 2
    rps = npad // NS

    @functools.partial(
        pl.kernel,
        out_type=jax.ShapeDtypeStruct((NC, npad, d), jnp.float32),
        mesh=_sc_mesh(),
        scratch_types=[
            pltpu.VMEM((half, CHUNK), jnp.int32),
            pltpu.VMEM((half, CHUNK), jnp.int32),
            pltpu.VMEM((CHUNK, d), jnp.float32),
            pltpu.VMEM((CHUNK, d), jnp.float32),
            pltpu.VMEM_SHARED((npad, d), jnp.float32),
            pltpu.SemaphoreType.DMA,
            pltpu.SemaphoreType.DMA,
        ],
    )
    def agg_kernel(y_hbm, src_hbm, dst_hbm, zeros_hbm, out_hbm,
                   isrc_v, idst_v, rows0_v, rows1_v, acc_sh, sem0, sem1):
        c = lax.axis_index("c")
        s = lax.axis_index("s")
        wid = c * NS + s
        pltpu.sync_copy(
            zeros_hbm.at[pl.ds(s * rps, rps)], acc_sh.at[pl.ds(s * rps, rps)]
        )
        plsc.subcore_barrier()

        def body(i, carry):
            pltpu.async_copy(y_hbm.at[isrc_v.at[2 * i]], rows0_v, sem0).wait()
            d1 = pltpu.async_copy(y_hbm.at[isrc_v.at[2 * i + 1]], rows1_v, sem1)
            pltpu.sync_copy(rows0_v, acc_sh.at[idst_v.at[2 * i]], add=True)
            d1.wait()
            pltpu.sync_copy(rows1_v, acc_sh.at[idst_v.at[2 * i + 1]], add=True)
            return carry

        for phase in range(2):
            pltpu.sync_copy(src_hbm.at[wid].at[pl.ds(phase * half, half)], isrc_v)
            pltpu.sync_copy(dst_hbm.at[wid].at[pl.ds(phase * half, half)], idst_v)
            lax.fori_loop(0, half // 2, body, 0)
        plsc.subcore_barrier()
        pltpu.sync_copy(
            acc_sh.at[pl.ds(s * rps, rps)], out_hbm.at[c, pl.ds(s * rps, rps)]
        )

    return agg_kernel(y, src3d, dst3d, zeros_nd)


def _tc_prepare(x, w1, degp):
    n, d = x.shape[0], w1.shape[1]

    def body(x_ref, w_ref, deg_ref, y_ref, dinv_ref):
        deg = deg_ref[0, :n, 0] + deg_ref[1, :n, 0] + 1.0
        dinv = lax.rsqrt(deg)
        dinv_ref[0, :] = dinv
        xw = jnp.dot(x_ref[...], w_ref[...], preferred_element_type=jnp.float32)
        y_ref[...] = xw * dinv[:, None]

    return pl.pallas_call(
        body,
        out_shape=(
            jax.ShapeDtypeStruct((n, d), jnp.float32),
            jax.ShapeDtypeStruct((1, n), jnp.float32),
        ),
    )(x, w1, degp)


def _tc_mid(aggp, y1, dinv, b1, w2):
    n, d = y1.shape

    def body(agg_ref, y1_ref, dinv_ref, b1_ref, w2_ref, y2_ref):
        dinv_col = dinv_ref[0, :][:, None]
        h = (agg_ref[0, :n] + agg_ref[1, :n] + y1_ref[...]) * dinv_col + b1_ref[...]
        h = jnp.maximum(h, 0.0)
        hw = jnp.dot(h, w2_ref[...], preferred_element_type=jnp.float32)
        y2_ref[...] = hw * dinv_col

    return pl.pallas_call(
        body, out_shape=jax.ShapeDtypeStruct((n, d), jnp.float32)
    )(aggp, y1, dinv, b1, w2)


def _tc_final(aggp, y2, dinv, b2, batch_row, n_graphs):
    n, d = y2.shape

    def body(agg_ref, y2_ref, dinv_ref, b2_ref, batch_ref, out_ref):
        dinv_col = dinv_ref[0, :][:, None]
        z = (agg_ref[0, :n] + agg_ref[1, :n] + y2_ref[...]) * dinv_col + b2_ref[...]
        gids = lax.broadcasted_iota(jnp.int32, (n_graphs, n), 0)
        onehot = (gids == batch_ref[0, :][None, :]).astype(jnp.float32)
        sums = jnp.dot(onehot, z, preferred_element_type=jnp.float32)
        cnt = jnp.sum(onehot, axis=1, keepdims=True)
        out_ref[...] = sums / jnp.maximum(cnt, 1.0)

    return pl.pallas_call(
        body, out_shape=jax.ShapeDtypeStruct((n_graphs, d), jnp.float32)
    )(aggp, y2, dinv, b2, batch_row)


def kernel(x, edge_index, batch, W1, b1, W2, b2):
    n, d = x.shape
    e = edge_index.shape[1]
    n_graphs = 32
    npad = ((n + NS * 8 - 1) // (NS * 8)) * NS * 8

    quantum = NW * CHUNK
    epad = ((e + 2 * quantum - 1) // (2 * quantum)) * 2 * quantum
    src_p = jnp.concatenate(
        [edge_index[0], jnp.zeros((epad - e,), edge_index.dtype)])
    pad_dst = n + jnp.arange(epad - e, dtype=edge_index.dtype) % (npad - n)
    dst_p = jnp.concatenate([edge_index[1], pad_dst])
    src3d = src_p.reshape(NW, epad // (NW * CHUNK), CHUNK)
    dst3d = dst_p.reshape(NW, epad // (NW * CHUNK), CHUNK)
    zeros16 = jnp.zeros((npad, 16), jnp.float32)
    zeros_nd = jnp.zeros((npad, d), jnp.float32)
    b1r = b1.reshape(1, d)
    b2r = b2.reshape(1, d)
    batch_row = batch.reshape(1, n).astype(jnp.int32)

    degp = _sc_degree(dst3d, zeros16, npad)
    y1, dinv = _tc_prepare(x, W1, degp)
    agg1 = _sc_aggregate(y1, src3d, dst3d, zeros_nd, npad)
    y2 = _tc_mid(agg1, y1, dinv, b1r, W2)
    agg2 = _sc_aggregate(y2, src3d, dst3d, zeros_nd, npad)
    return _tc_final(agg2, y2, dinv, b2r, batch_row, n_graphs)


# speedup vs baseline: 3.2722x; 3.2722x over previous
import functools

import jax
import jax.numpy as jnp
from jax import lax
from jax.experimental import pallas as pl
from jax.experimental.pallas import tpu as pltpu
from jax.experimental.pallas import tpu_sc as plsc

NC = 2
NS = 16
NW = NC * NS
CHUNK = 125


def _sc_mesh():
    return plsc.VectorSubcoreMesh(
        core_axis_name="c", subcore_axis_name="s", num_cores=NC, num_subcores=NS
    )


def _sc_degree(dst3d, zeros16, npad):
    cpw = dst3d.shape[1]
    half = cpw // 2
    rps = npad // NS

    @functools.partial(
        pl.kernel,
        out_type=jax.ShapeDtypeStruct((NC, npad, 16), jnp.float32),
        mesh=_sc_mesh(),
        scratch_types=[
            pltpu.VMEM((half, CHUNK), jnp.int32),
            pltpu.VMEM((CHUNK, 16), jnp.float32),
            pltpu.VMEM_SHARED((npad, 16), jnp.float32),
        ],
    )
    def deg_kernel(dst_hbm, zeros_hbm, out_hbm, idx_v, ones_v, acc_sh):
        c = lax.axis_index("c")
        s = lax.axis_index("s")
        wid = c * NS + s

        def fill(i, carry):
            ones_v[i, :] = jnp.full((16,), 1.0, jnp.float32)
            return carry

        lax.fori_loop(0, CHUNK, fill, 0)
        pltpu.sync_copy(
            zeros_hbm.at[pl.ds(s * rps, rps)], acc_sh.at[pl.ds(s * rps, rps)]
        )
        plsc.subcore_barrier()

        def body(j, carry):
            pltpu.sync_copy(ones_v, acc_sh.at[idx_v.at[j]], add=True)
            return carry

        for phase in range(2):
            pltpu.sync_copy(dst_hbm.at[wid].at[pl.ds(phase * half, half)], idx_v)
            lax.fori_loop(0, half, body, 0)
        plsc.subcore_barrier()
        pltpu.sync_copy(
            acc_sh.at[pl.ds(s * rps, rps)], out_hbm.at[c, pl.ds(s * rps, rps)]
        )

    return deg_kernel(dst3d, zeros16)


def _sc_aggregate(y, src3d, dst3d, zeros_nd, npad):
    d = y.shape[1]
    cpw = src3d.shape[1]
    half = cpw // 2
    rps = npad // NS

    @functools.partial(
        pl.kernel,
        out_type=jax.ShapeDtypeStruct((NC, npad, d), jnp.float32),
        mesh=_sc_mesh(),
        scratch_types=[
            pltpu.VMEM((half, CHUNK), jnp.int32),
            pltpu.VMEM((half, CHUNK), jnp.int32),
            pltpu.VMEM((CHUNK, d), jnp.float32),
            pltpu.VMEM((CHUNK, d), jnp.float32),
            pltpu.VMEM_SHARED((npad, d), jnp.float32),
            pltpu.SemaphoreType.DMA,
            pltpu.SemaphoreType.DMA,
        ],
    )
    def agg_kernel(y_hbm, src_hbm, dst_hbm, zeros_hbm, out_hbm,
                   isrc_v, idst_v, rows0_v, rows1_v, acc_sh, sem0, sem1):
        c = lax.axis_index("c")
        s = lax.axis_index("s")
        wid = c * NS + s
        pltpu.sync_copy(
            zeros_hbm.at[pl.ds(s * rps, rps)], acc_sh.at[pl.ds(s * rps, rps)]
        )
        plsc.subcore_barrier()

        def body(i, carry):
            pltpu.async_copy(y_hbm.at[isrc_v.at[2 * i]], rows0_v, sem0).wait()
            d1 = pltpu.async_copy(y_hbm.at[isrc_v.at[2 * i + 1]], rows1_v, sem1)
            pltpu.sync_copy(rows0_v, acc_sh.at[idst_v.at[2 * i]], add=True)
            d1.wait()
            pltpu.sync_copy(rows1_v, acc_sh.at[idst_v.at[2 * i + 1]], add=True)
            return carry

        for phase in range(2):
            pltpu.sync_copy(src_hbm.at[wid].at[pl.ds(phase * half, half)], isrc_v)
            pltpu.sync_copy(dst_hbm.at[wid].at[pl.ds(phase * half, half)], idst_v)
            lax.fori_loop(0, half // 2, body, 0)
        plsc.subcore_barrier()
        pltpu.sync_copy(
            acc_sh.at[pl.ds(s * rps, rps)], out_hbm.at[c, pl.ds(s * rps, rps)]
        )

    return agg_kernel(y, src3d, dst3d, zeros_nd)


def _tc_prepare(x, w1, degp):
    n, d = x.shape[0], w1.shape[1]

    def body(x_ref, w_ref, deg_ref, y_ref, dinv_ref):
        deg = deg_ref[0, :n, 0] + deg_ref[1, :n, 0] + 1.0
        dinv = lax.rsqrt(deg)
        dinv_ref[0, :] = dinv
        xw = jnp.dot(x_ref[...], w_ref[...], preferred_element_type=jnp.float32)
        y_ref[...] = xw * dinv[:, None]

    return pl.pallas_call(
        body,
        out_shape=(
            jax.ShapeDtypeStruct((n, d), jnp.float32),
            jax.ShapeDtypeStruct((1, n), jnp.float32),
        ),
    )(x, w1, degp)


def _tc_mid(aggp, y1, dinv, b1, w2):
    n, d = y1.shape

    def body(agg_ref, y1_ref, dinv_ref, b1_ref, w2_ref, y2_ref):
        dinv_col = dinv_ref[0, :][:, None]
        h = (agg_ref[0, :n] + agg_ref[1, :n] + y1_ref[...]) * dinv_col + b1_ref[...]
        h = jnp.maximum(h, 0.0)
        hw = jnp.dot(h, w2_ref[...], preferred_element_type=jnp.float32)
        y2_ref[...] = hw * dinv_col

    return pl.pallas_call(
        body, out_shape=jax.ShapeDtypeStruct((n, d), jnp.float32)
    )(aggp, y1, dinv, b1, w2)


def _tc_final(aggp, y2, dinv, b2, batch_row, n_graphs):
    n, d = y2.shape

    def body(agg_ref, y2_ref, dinv_ref, b2_ref, batch_ref, out_ref):
        dinv_col = dinv_ref[0, :][:, None]
        z = (agg_ref[0, :n] + agg_ref[1, :n] + y2_ref[...]) * dinv_col + b2_ref[...]
        gids = lax.broadcasted_iota(jnp.int32, (n_graphs, n), 0)
        onehot = (gids == batch_ref[0, :][None, :]).astype(jnp.float32)
        sums = jnp.dot(onehot, z, preferred_element_type=jnp.float32)
        cnt = jnp.sum(onehot, axis=1, keepdims=True)
        out_ref[...] = sums / jnp.maximum(cnt, 1.0)

    return pl.pallas_call(
        body, out_shape=jax.ShapeDtypeStruct((n_graphs, d), jnp.float32)
    )(aggp, y2, dinv, b2, batch_row)


def kernel(x, edge_index, batch, W1, b1, W2, b2):
    n, d = x.shape
    e = edge_index.shape[1]
    n_graphs = 32
    npad = ((n + NS * 8 - 1) // (NS * 8)) * NS * 8

    quantum = NW * CHUNK
    epad = ((e + 2 * quantum - 1) // (2 * quantum)) * 2 * quantum
    src_p = jnp.concatenate(
        [edge_index[0], jnp.zeros((epad - e,), edge_index.dtype)])
    pad_dst = n + jnp.arange(epad - e, dtype=edge_index.dtype) % (npad - n)
    dst_p = jnp.concatenate([edge_index[1], pad_dst])
    src3d = src_p.reshape(NW, epad // (NW * CHUNK), CHUNK)
    dst3d = dst_p.reshape(NW, epad // (NW * CHUNK), CHUNK)
    zeros16 = jnp.zeros((npad, 16), jnp.float32)
    zeros_nd = jnp.zeros((npad, d), jnp.float32)
    b1r = b1.reshape(1, d)
    b2r = b2.reshape(1, d)
    batch_row = batch.reshape(1, n).astype(jnp.int32)

    degp = _sc_degree(dst3d, zeros16, npad)
    y1, dinv = _tc_prepare(x, W1, degp)
    agg1 = _sc_aggregate(y1, src3d, dst3d, zeros_nd, npad)
    y2 = _tc_mid(agg1, y1, dinv, b1r, W2)
    agg2 = _sc_aggregate(y2, src3d, dst3d, zeros_nd, npad)
    return _tc_final(agg2, y2, dinv, b2r, batch_row, n_graphs)
